# Initial kernel scaffold; baseline (speedup 1.0000x reference)
import jax
import jax.numpy as jnp
from jax.experimental import pallas as pl


def kernel(x, weight):
    raise NotImplementedError("write your pallas kernel here")


# speedup vs baseline: 1.3825x; 1.3825x over previous
import functools

import jax
import jax.numpy as jnp
from jax import lax
from jax.experimental import pallas as pl
from jax.experimental.pallas import tpu as pltpu
from jax.experimental.pallas import tpu_sc as plsc

_NUM_EMB = 1000000
_DIM = 64
_BATCH = 16384
_HIST = 50
_N_IDX = _BATCH * _HIST

_MM_BLOCK = 8000

_ROUND_MAGIC = 12582912.0


def _minmax_body(w_ref, o_ref):
    i = pl.program_id(0)
    bmin = jnp.min(w_ref[...])
    bmax = jnp.max(w_ref[...])

    @pl.when(i == 0)
    def _init():
        o_ref[0] = bmin
        o_ref[1] = bmax

    @pl.when(i > 0)
    def _acc():
        o_ref[0] = jnp.minimum(o_ref[0], bmin)
        o_ref[1] = jnp.maximum(o_ref[1], bmax)


def _table_minmax(weight):
    return pl.pallas_call(
        _minmax_body,
        grid=(_NUM_EMB // _MM_BLOCK,),
        in_specs=[pl.BlockSpec((_MM_BLOCK, _DIM), lambda i: (i, 0))],
        out_specs=pl.BlockSpec(memory_space=pltpu.SMEM),
        out_shape=jax.ShapeDtypeStruct((2,), jnp.float32),
    )(weight)


def _make_gather_kernel(n_workers, b_per_w, chunk):
    n_chunks = b_per_w // chunk
    mesh = plsc.VectorSubcoreMesh(core_axis_name="c", subcore_axis_name="s")

    @functools.partial(
        pl.kernel,
        mesh=mesh,
        compiler_params=pltpu.CompilerParams(use_tc_tiling_on_sc=False),
        out_type=jax.ShapeDtypeStruct((_N_IDX, _DIM), jnp.float32),
        scratch_types=[
            pltpu.VMEM((chunk,), jnp.int32),
            pltpu.VMEM((chunk, _DIM), jnp.float32),
            pltpu.VMEM((4, 16), jnp.float32),
            pltpu.SemaphoreType.DMA,
        ],
    )
    def gather_dequant(idx_hbm, table_hbm, params_hbm, out_hbm,
                       idx_v, rows_v, params_v, sem):
        wid = lax.axis_index("s") * 2 + lax.axis_index("c")
        base = wid * b_per_w
        pltpu.sync_copy(params_hbm, params_v)
        inv_scale = params_v[0, :]
        zp = params_v[1, :]
        scale = params_v[2, :]
        zp_scale = params_v[3, :]

        def chunk_body(k, carry):
            off = base + k * chunk
            pltpu.sync_copy(idx_hbm.at[pl.ds(off, chunk)], idx_v)
            pltpu.async_copy(table_hbm.at[idx_v], rows_v, sem).wait()

            def row_body(r, c2):
                for j in range(_DIM // 16):
                    w = rows_v[r, pl.ds(j * 16, 16)]
                    t = w * inv_scale + zp
                    t = jnp.maximum(t, -128.0)
                    t = jnp.minimum(t, 127.0)
                    q = (t + _ROUND_MAGIC) - _ROUND_MAGIC
                    rows_v[r, pl.ds(j * 16, 16)] = q * scale - zp_scale
                return c2

            lax.fori_loop(0, chunk, row_body, 0)
            pltpu.sync_copy(rows_v, out_hbm.at[pl.ds(off, chunk)])
            return carry

        lax.fori_loop(0, n_chunks, chunk_body, 0)

    return gather_dequant


def kernel(x, weight):
    mm = _table_minmax(weight)
    wmin, wmax = mm[0], mm[1]
    scale = (wmax - wmin) / 255.0
    zp = -128.0 - wmin / scale
    params = jnp.stack([
        jnp.full((16,), 1.0 / scale, jnp.float32),
        jnp.full((16,), zp, jnp.float32),
        jnp.full((16,), scale, jnp.float32),
        jnp.full((16,), zp * scale, jnp.float32),
    ])

    info = plsc.get_sparse_core_info()
    n_workers = info.num_cores * info.num_subcores
    b_per_w = _N_IDX // n_workers
    k = _make_gather_kernel(n_workers, b_per_w, chunk=1600)
    out2d = k(x.reshape(-1), weight, params)
    return out2d.reshape(_BATCH, _HIST, _DIM)
